# baseline (speedup 1.0000x reference)
import functools

import jax
import jax.numpy as jnp
from jax import lax
from jax.experimental import pallas as pl
from jax.experimental.pallas import tpu as pltpu
from jax.experimental.pallas import tpu_sc as plsc

VOCAB = 1000000
EMBED_DIM = 32
BATCH = 16384
HIST = 50
NUM_WORKERS = 32
CHUNK = 128
B_PER_W = BATCH // NUM_WORKERS
BLK_PER_W = B_PER_W // CHUNK
U_PER_W = HIST * BLK_PER_W
NBUF = 4
PITCH = CHUNK + 1
TPITCH = EMBED_DIM + 1

N_FULL = VOCAB // CHUNK
T_LOOP = N_FULL // NUM_WORKERS
N_TAIL = N_FULL - T_LOOP * NUM_WORKERS
REM_V0 = N_FULL * CHUNK
REM = VOCAB - REM_V0

_mesh = plsc.VectorSubcoreMesh(core_axis_name="c", subcore_axis_name="s")
_params = pltpu.CompilerParams(
    use_tc_tiling_on_sc=False, needs_layout_passes=False)


@functools.partial(
    pl.kernel,
    mesh=_mesh,
    compiler_params=_params,
    out_type=jax.ShapeDtypeStruct((VOCAB, EMBED_DIM), jnp.float32),
    scratch_types=[
        pltpu.VMEM((NBUF, EMBED_DIM, CHUNK), jnp.float32),
        pltpu.VMEM((CHUNK, TPITCH), jnp.float32),
        pltpu.SemaphoreType.DMA((NBUF,)),
    ],
)
def _transpose_table(tt_hbm, out_hbm, cols_v, rows_v, gsem):
    wid = lax.axis_index("s") * 2 + lax.axis_index("c")
    iota = lax.iota(jnp.int32, 16)

    def fire(i, slot):
        c = i * NUM_WORKERS + wid
        pltpu.async_copy(tt_hbm.at[:, pl.ds(c * CHUNK, CHUNK)],
                         cols_v.at[slot], gsem.at[slot])

    def drain(i, slot):
        c = i * NUM_WORKERS + wid
        pltpu.make_async_copy(tt_hbm.at[:, pl.ds(c * CHUNK, CHUNK)],
                              cols_v.at[slot], gsem.at[slot]).wait()

    def transpose_slab(slot, n_groups):
        for g in range(n_groups):
            rows = iota + (g * 16)
            for d in range(EMBED_DIM):
                cols = jnp.full((16,), d, jnp.int32)
                plsc.store_scatter(
                    rows_v, [rows, cols],
                    cols_v[slot, d, pl.ds(g * 16, 16)])

    for i in range(NBUF):
        fire(i, i)

    @pl.loop(0, T_LOOP)
    def chunk(i):
        slot = lax.rem(i, NBUF)
        drain(i, slot)
        transpose_slab(slot, CHUNK // 16)

        @pl.when(i + NBUF < T_LOOP)
        def refire():
            fire(i + NBUF, slot)

        c = i * NUM_WORKERS + wid
        pltpu.sync_copy(rows_v.at[:, pl.ds(0, EMBED_DIM)],
                        out_hbm.at[pl.ds(c * CHUNK, CHUNK)])

    @pl.when(wid < N_TAIL)
    def tail_full():
        v0 = (T_LOOP * NUM_WORKERS + wid) * CHUNK
        pltpu.sync_copy(tt_hbm.at[:, pl.ds(v0, CHUNK)], cols_v.at[0])
        transpose_slab(0, CHUNK // 16)
        pltpu.sync_copy(rows_v.at[:, pl.ds(0, EMBED_DIM)],
                        out_hbm.at[pl.ds(v0, CHUNK)])

    @pl.when(wid == N_TAIL)
    def tail_rem():
        pltpu.sync_copy(tt_hbm.at[:, pl.ds(REM_V0, REM)],
                        cols_v.at[0, :, pl.ds(0, REM)])
        transpose_slab(0, REM // 16)
        pltpu.sync_copy(rows_v.at[pl.ds(0, REM), pl.ds(0, EMBED_DIM)],
                        out_hbm.at[pl.ds(REM_V0, REM)])


@functools.partial(
    pl.kernel,
    mesh=_mesh,
    compiler_params=_params,
    out_type=jax.ShapeDtypeStruct((HIST, EMBED_DIM, BATCH), jnp.float32),
    scratch_types=[
        pltpu.VMEM((HIST, B_PER_W), jnp.int32),
        pltpu.VMEM((NBUF, CHUNK, EMBED_DIM), jnp.float32),
        pltpu.VMEM((EMBED_DIM, PITCH), jnp.float32),
        pltpu.SemaphoreType.DMA((NBUF,)),
    ],
)
def _gather_one(idx_hbm, table_hbm, out_hbm, idx_v, rows_v, trans_v, gsem):
    wid = lax.axis_index("s") * 2 + lax.axis_index("c")
    b_base = wid * B_PER_W
    pltpu.sync_copy(idx_hbm.at[:, pl.ds(b_base, B_PER_W)], idx_v)

    def unit_idx(t):
        h = t // BLK_PER_W
        off = (t % BLK_PER_W) * CHUNK
        return h, off

    def fire(t, slot):
        h, off = unit_idx(t)
        isl = idx_v.at[h, pl.ds(off, CHUNK)]
        pltpu.async_copy(table_hbm.at[isl], rows_v.at[slot], gsem.at[slot])

    def drain(t, slot):
        h, off = unit_idx(t)
        isl = idx_v.at[h, pl.ds(off, CHUNK)]
        pltpu.make_async_copy(
            table_hbm.at[isl], rows_v.at[slot], gsem.at[slot]).wait()

    for t in range(NBUF):
        fire(t, t)

    iota = lax.iota(jnp.int32, 16)
    rows_lo = iota
    rows_hi = iota + 16

    @pl.loop(0, U_PER_W)
    def unit(t):
        slot = lax.rem(t, NBUF)
        drain(t, slot)
        for bb in range(CHUNK):
            cols = jnp.full((16,), bb, jnp.int32)
            plsc.store_scatter(
                trans_v, [rows_lo, cols], rows_v[slot, bb, pl.ds(0, 16)])
            plsc.store_scatter(
                trans_v, [rows_hi, cols], rows_v[slot, bb, pl.ds(16, 16)])

        @pl.when(t + NBUF < U_PER_W)
        def refire():
            fire(t + NBUF, slot)

        h, off = unit_idx(t)
        b0 = b_base + off
        pltpu.sync_copy(trans_v.at[:, pl.ds(0, CHUNK)],
                        out_hbm.at[h, :, pl.ds(b0, CHUNK)])


def kernel(inputs, loc_table, scale_table):
    idx_t = inputs.astype(jnp.int32).T
    loc_rm = _transpose_table(loc_table.T)
    scale_rm = _transpose_table(scale_table.T)
    out_loc = _gather_one(idx_t, loc_rm)
    out_scale = _gather_one(idx_t, scale_rm)
    return (jnp.transpose(out_loc, (2, 0, 1)),
            jnp.transpose(out_scale, (2, 0, 1)))


# speedup vs baseline: 4.8416x; 4.8416x over previous
import functools

import jax
import jax.numpy as jnp
from jax import lax
from jax.experimental import pallas as pl
from jax.experimental.pallas import tpu as pltpu
from jax.experimental.pallas import tpu_sc as plsc

EMBED_DIM = 32
BATCH = 16384
HIST = 50
NUM_WORKERS = 32
CHUNK = 128
B_PER_W = BATCH // NUM_WORKERS
BLK_PER_W = B_PER_W // CHUNK
U_PER_W = HIST * BLK_PER_W
NBUF = 4
PITCH = CHUNK + 1

_mesh = plsc.VectorSubcoreMesh(core_axis_name="c", subcore_axis_name="s")


@functools.partial(
    pl.kernel,
    mesh=_mesh,
    compiler_params=pltpu.CompilerParams(
        use_tc_tiling_on_sc=False, needs_layout_passes=False),
    out_type=jax.ShapeDtypeStruct((HIST, EMBED_DIM, BATCH), jnp.float32),
    scratch_types=[
        pltpu.VMEM((HIST, B_PER_W), jnp.int32),
        pltpu.VMEM((NBUF, CHUNK, EMBED_DIM), jnp.float32),
        pltpu.VMEM((EMBED_DIM, PITCH), jnp.float32),
        pltpu.SemaphoreType.DMA((NBUF,)),
    ],
)
def _gather_one(idx_hbm, table_hbm, out_hbm, idx_v, rows_v, trans_v, gsem):
    wid = lax.axis_index("s") * 2 + lax.axis_index("c")
    b_base = wid * B_PER_W
    pltpu.sync_copy(idx_hbm.at[:, pl.ds(b_base, B_PER_W)], idx_v)

    def unit_idx(t):
        h = t // BLK_PER_W
        off = (t % BLK_PER_W) * CHUNK
        return h, off

    def fire(t, slot):
        h, off = unit_idx(t)
        isl = idx_v.at[h, pl.ds(off, CHUNK)]
        pltpu.async_copy(table_hbm.at[isl], rows_v.at[slot], gsem.at[slot])

    def drain(t, slot):
        h, off = unit_idx(t)
        isl = idx_v.at[h, pl.ds(off, CHUNK)]
        pltpu.make_async_copy(
            table_hbm.at[isl], rows_v.at[slot], gsem.at[slot]).wait()

    for t in range(NBUF):
        fire(t, t)

    iota = lax.iota(jnp.int32, 16)
    rows_lo = iota
    rows_hi = iota + 16

    @pl.loop(0, U_PER_W)
    def unit(t):
        slot = lax.rem(t, NBUF)
        drain(t, slot)
        for bb in range(CHUNK):
            cols = jnp.full((16,), bb, jnp.int32)
            plsc.store_scatter(
                trans_v, [rows_lo, cols], rows_v[slot, bb, pl.ds(0, 16)])
            plsc.store_scatter(
                trans_v, [rows_hi, cols], rows_v[slot, bb, pl.ds(16, 16)])

        @pl.when(t + NBUF < U_PER_W)
        def refire():
            fire(t + NBUF, slot)

        h, off = unit_idx(t)
        b0 = b_base + off
        pltpu.sync_copy(trans_v.at[:, pl.ds(0, CHUNK)],
                        out_hbm.at[h, :, pl.ds(b0, CHUNK)])


def kernel(inputs, loc_table, scale_table):
    idx_t = inputs.astype(jnp.int32).T
    out_loc = _gather_one(idx_t, loc_table)
    out_scale = _gather_one(idx_t, scale_table)
    return (jnp.transpose(out_loc, (2, 0, 1)),
            jnp.transpose(out_scale, (2, 0, 1)))
